# baseline (speedup 1.0000x reference)
import functools

import jax
import jax.numpy as jnp
from jax import lax
from jax.experimental import pallas as pl
from jax.experimental.pallas import tpu as pltpu
from jax.experimental.pallas import tpu_sc as plsc

N = 2048
D = 768
DFF = 2048
E = 8
R = 16

BT = 256
NB = 2 * N // BT + E
S = NB * BT

NW = 32
SPT = S // NW
PPT = 2 * N // NW
TPT = N // NW
GB = 64

_SC_MESH = dict(core_axis_name="c", subcore_axis_name="s")


def _logits_kernel(x_ref, wg_ref, lt_ref):
    lt_ref[...] = jax.lax.dot_general(
        wg_ref[...], x_ref[...], (((1,), (1,)), ((), ())),
        preferred_element_type=jnp.float32)


def _router_logits(x, Wg):
    return pl.pallas_call(
        _logits_kernel,
        out_shape=jax.ShapeDtypeStruct((E, N), jnp.float32),
    )(x, Wg)


NT16 = 16
TB = N // NT16
PB = 2 * TB


def _bucket_body(lt_hbm, rt_hbm, be_hbm, so_hbm, tw_hbm,
                 lt_v, w2_v, be_v, zero_v, tok2_v, pos2_v, cnt_sh, cnt_v,
                 cnt_all_v, sem):
    cid = lax.axis_index("c")
    sid = lax.axis_index("s")
    iota16 = lax.iota(jnp.int32, 16)

    @pl.when(cid == 0)
    def _bucket():
        t0 = sid * TB

        for j in range(S // NT16 // 16):
            zero_v[pl.ds(j * 16, 16)] = jnp.zeros((16,), jnp.int32)
        pltpu.sync_copy(zero_v, rt_hbm.at[pl.ds(sid * (S // NT16), S // NT16)])

        pltpu.sync_copy(lt_hbm.at[:, pl.ds(t0, TB)], lt_v)

        def route_chunk(c):
            base = c * 16
            ls = [lt_v[e, pl.ds(base, 16)] for e in range(E)]
            m1 = ls[0]
            i1 = jnp.zeros((16,), jnp.int32)
            for e in range(1, E):
                upd = ls[e] > m1
                m1 = jnp.where(upd, ls[e], m1)
                i1 = jnp.where(upd, e, i1)
            m2 = jnp.full((16,), -1e30, jnp.float32)
            i2 = jnp.zeros((16,), jnp.int32)
            for e in range(E):
                upd = (ls[e] > m2) & (i1 != e)
                m2 = jnp.where(upd, ls[e], m2)
                i2 = jnp.where(upd, e, i2)
            w1 = 1.0 / (1.0 + jnp.exp(m2 - m1))
            tok2_v[0, pl.ds(base, 16)] = i1
            tok2_v[1, pl.ds(base, 16)] = i2
            return w1

        for c in range(TB // 16):
            w1 = route_chunk(c)
            w2_v[0, pl.ds(c * 16, 16)] = w1
            w2_v[1, pl.ds(c * 16, 16)] = 1.0 - w1
        pltpu.sync_copy(w2_v.at[0], tw_hbm.at[pl.ds(t0, TB)])
        pltpu.sync_copy(w2_v.at[1], tw_hbm.at[pl.ds(N + t0, TB)])

        def count_chunk(c, cnt):
            for k in range(2):
                v = tok2_v[k, pl.ds(c * 16, 16)]
                for e in range(E):
                    pc = jnp.sum(jnp.where(v == e, 1, 0))
                    cnt = cnt + jnp.where(iota16 == e, pc, 0)
            return cnt

        cnt = lax.fori_loop(0, TB // 16, count_chunk,
                            jnp.zeros((16,), jnp.int32), unroll=False)
        cnt_v[pl.ds(0, 16)] = cnt
        pltpu.sync_copy(cnt_v, cnt_sh.at[pl.ds(sid * 16, 16)])
        plsc.subcore_barrier()

        pltpu.sync_copy(cnt_sh, cnt_all_v)
        totals = jnp.zeros((16,), jnp.int32)
        prefix = jnp.zeros((16,), jnp.int32)
        for t in range(NT16):
            row = cnt_all_v[pl.ds(t * 16, 16)]
            totals = totals + row
            prefix = prefix + jnp.where(t < sid, row, 0)
        nblk = (totals + (BT - 1)) // BT
        csum = plsc.cumsum(nblk)
        startblk = csum - nblk
        segstart = startblk * BT
        base_v = segstart + prefix

        @pl.when(sid == 0)
        def _():
            for j in range(2):
                bid = iota16 + 16 * j
                be = jnp.zeros((16,), jnp.int32)
                for e in range(E):
                    sb = jnp.sum(jnp.where(iota16 == e, startblk, 0))
                    eb = jnp.sum(jnp.where(iota16 == e, csum, 0))
                    be = jnp.where((bid >= sb) & (bid < eb), e, be)
                be_v[pl.ds(16 * j, 16)] = be
            pltpu.sync_copy(be_v, be_hbm)

        def pos_chunk(c, base_v):
            for k in range(2):
                v = tok2_v[k, pl.ds(c * 16, 16)]
                pos = jnp.zeros((16,), jnp.int32)
                for e in range(E):
                    m = v == e
                    inc = jnp.where(m, 1, 0)
                    s = plsc.cumsum(inc)
                    base_e = jnp.sum(jnp.where(iota16 == e, base_v, 0))
                    pos = jnp.where(m, base_e + s - 1, pos)
                    base_v = base_v + jnp.where(iota16 == e, jnp.sum(inc), 0)
                pos2_v[k, pl.ds(c * 16, 16)] = pos
            return base_v

        lax.fori_loop(0, TB // 16, pos_chunk, base_v, unroll=False)

        for c in range(TB // 16):
            tid = t0 + c * 16 + iota16
            tok2_v[0, pl.ds(c * 16, 16)] = tid
            tok2_v[1, pl.ds(c * 16, 16)] = tid

        pltpu.sync_copy(pos2_v.at[0], so_hbm.at[pl.ds(t0, TB)])
        pltpu.sync_copy(pos2_v.at[1], so_hbm.at[pl.ds(N + t0, TB)])

        plsc.subcore_barrier()
        pltpu.sync_copy(tok2_v.at[0], rt_hbm.at[pos2_v.at[0]])
        pltpu.sync_copy(tok2_v.at[1], rt_hbm.at[pos2_v.at[1]])


def _bucketize(lt2d):
    mesh = plsc.VectorSubcoreMesh(**_SC_MESH)
    fn = functools.partial(
        pl.kernel,
        out_type=[
            jax.ShapeDtypeStruct((S,), jnp.int32),
            jax.ShapeDtypeStruct((NW,), jnp.int32),
            jax.ShapeDtypeStruct((2 * N,), jnp.int32),
            jax.ShapeDtypeStruct((2 * N,), jnp.float32),
        ],
        mesh=mesh,
        compiler_params=pltpu.CompilerParams(needs_layout_passes=False),
        scratch_types=[
            pltpu.VMEM((E, TB), jnp.float32),
            pltpu.VMEM((2, TB), jnp.float32),
            pltpu.VMEM((NW,), jnp.int32),
            pltpu.VMEM((S // NT16,), jnp.int32),
            pltpu.VMEM((2, TB), jnp.int32),
            pltpu.VMEM((2, TB), jnp.int32),
            pltpu.VMEM_SHARED((NT16 * 16,), jnp.int32),
            pltpu.VMEM((16,), jnp.int32),
            pltpu.VMEM((NT16 * 16,), jnp.int32),
            pltpu.SemaphoreType.DMA,
        ],
    )(_bucket_body)
    return fn(lt2d)


D2 = D // 2


def _gather_body(rt_hbm, x_hbm, xs_hbm, idx_v, buf0, buf1, buf2,
                 sem0, sem1, sem2):
    cid = lax.axis_index("c")
    sid = lax.axis_index("s")
    wid = sid * 2 + cid
    lo = wid * SPT

    pltpu.sync_copy(rt_hbm.at[pl.ds(lo, SPT)], idx_v)
    bufs = [buf0, buf1, buf2]
    sems = [sem0, sem1, sem2]
    cps = [pltpu.async_copy(x_hbm.at[idx_v.at[pl.ds(j * GB, GB)]],
                            bufs[j], sems[j])
           for j in range(SPT // GB)]
    for j in range(SPT // GB):
        cps[j].wait()
        pltpu.sync_copy(bufs[j], xs_hbm.at[pl.ds(lo + j * GB, GB)])


def _gather_x(rt, x_i32):
    mesh = plsc.VectorSubcoreMesh(**_SC_MESH)
    fn = functools.partial(
        pl.kernel,
        out_type=jax.ShapeDtypeStruct((S, D2), jnp.int32),
        mesh=mesh,
        compiler_params=pltpu.CompilerParams(needs_layout_passes=False),
        scratch_types=[
            pltpu.VMEM((SPT,), jnp.int32),
            pltpu.VMEM((GB, D2), jnp.int32),
            pltpu.VMEM((GB, D2), jnp.int32),
            pltpu.VMEM((GB, D2), jnp.int32),
            pltpu.SemaphoreType.DMA,
            pltpu.SemaphoreType.DMA,
            pltpu.SemaphoreType.DMA,
        ],
    )(_gather_body)
    return fn(rt, x_i32)


def _bdot_t(a, b):
    return jax.lax.dot_general(a, b, (((1,), (1,)), ((), ())),
                               preferred_element_type=jnp.float32)


def _grouped_kernel(be_ref, xs_ref, w1_ref, w3_ref, w2_ref,
                    a1_ref, b1_ref, a3_ref, b3_ref, a2_ref, b2_ref,
                    a1b_ref, b1b_ref, a3b_ref, b3b_ref, a2b_ref, b2b_ref,
                    out_ref):
    las = [(a1_ref, b1_ref, a3_ref, b3_ref, a2_ref, b2_ref),
           (a1b_ref, b1b_ref, a3b_ref, b3b_ref, a2b_ref, b2b_ref)]
    for i in range(2):
        a1r, b1r, a3r, b3r, a2r, b2r = las[i]
        rows = pl.ds(i * BT, BT)
        xb = xs_ref[rows, :]
        u1 = _bdot_t(xb, a1r[0]).astype(jnp.bfloat16)
        w1 = _bdot_t(xb, w1_ref[...]) + _bdot_t(u1, b1r[0])
        u3 = _bdot_t(xb, a3r[0]).astype(jnp.bfloat16)
        w3 = _bdot_t(xb, w3_ref[...]) + _bdot_t(u3, b3r[0])
        h = (w1 * jax.nn.sigmoid(w1) * w3).astype(jnp.bfloat16)
        u2 = _bdot_t(h, a2r[0]).astype(jnp.bfloat16)
        out_ref[rows, :] = _bdot_t(h, w2_ref[...]) + _bdot_t(u2, b2r[0])


def _grouped_ffn(be, xs, W1b, W3b, W2b, A1b, B1b, A3b, B3b, A2b, B2b):
    grid_spec = pltpu.PrefetchScalarGridSpec(
        num_scalar_prefetch=1,
        grid=(NB // 2,),
        in_specs=[
            pl.BlockSpec((2 * BT, D), lambda b, be: (b, 0)),
            pl.BlockSpec((DFF, D), lambda b, be: (0, 0)),
            pl.BlockSpec((DFF, D), lambda b, be: (0, 0)),
            pl.BlockSpec((D, DFF), lambda b, be: (0, 0)),
            pl.BlockSpec((1, R, D), lambda b, be: (be[2 * b], 0, 0)),
            pl.BlockSpec((1, DFF, R), lambda b, be: (be[2 * b], 0, 0)),
            pl.BlockSpec((1, R, D), lambda b, be: (be[2 * b], 0, 0)),
            pl.BlockSpec((1, DFF, R), lambda b, be: (be[2 * b], 0, 0)),
            pl.BlockSpec((1, R, DFF), lambda b, be: (be[2 * b], 0, 0)),
            pl.BlockSpec((1, D, R), lambda b, be: (be[2 * b], 0, 0)),
            pl.BlockSpec((1, R, D), lambda b, be: (be[2 * b + 1], 0, 0)),
            pl.BlockSpec((1, DFF, R), lambda b, be: (be[2 * b + 1], 0, 0)),
            pl.BlockSpec((1, R, D), lambda b, be: (be[2 * b + 1], 0, 0)),
            pl.BlockSpec((1, DFF, R), lambda b, be: (be[2 * b + 1], 0, 0)),
            pl.BlockSpec((1, R, DFF), lambda b, be: (be[2 * b + 1], 0, 0)),
            pl.BlockSpec((1, D, R), lambda b, be: (be[2 * b + 1], 0, 0)),
        ],
        out_specs=pl.BlockSpec((2 * BT, D), lambda b, be: (b, 0)),
    )
    return pl.pallas_call(
        _grouped_kernel,
        grid_spec=grid_spec,
        out_shape=jax.ShapeDtypeStruct((S, D), jnp.float32),
        compiler_params=pltpu.CompilerParams(
            dimension_semantics=("parallel",),
        ),
    )(be, xs, W1b, W3b, W2b, A1b, B1b, A3b, B3b, A2b, B2b,
      A1b, B1b, A3b, B3b, A2b, B2b)


def _combine_body(os_hbm, so_hbm, tw_hbm, fin_hbm,
                  so_v, tw_v, r0_v, r1_v, sem):
    cid = lax.axis_index("c")
    sid = lax.axis_index("s")
    wid = sid * 2 + cid
    t0 = wid * TPT

    pltpu.sync_copy(so_hbm.at[pl.ds(t0, TPT)], so_v.at[pl.ds(0, TPT)])
    pltpu.sync_copy(so_hbm.at[pl.ds(N + t0, TPT)], so_v.at[pl.ds(TPT, TPT)])
    pltpu.sync_copy(tw_hbm.at[pl.ds(t0, TPT)], tw_v.at[pl.ds(0, TPT)])
    pltpu.sync_copy(tw_hbm.at[pl.ds(N + t0, TPT)], tw_v.at[pl.ds(TPT, TPT)])

    pltpu.async_copy(os_hbm.at[so_v.at[pl.ds(0, TPT)]], r0_v, sem).wait()
    pltpu.async_copy(os_hbm.at[so_v.at[pl.ds(TPT, TPT)]], r1_v, sem).wait()

    def tok_loop(i, carry):
        w0 = tw_v[pl.ds(i, 16)][0]
        w1 = tw_v[pl.ds(TPT + i, 16)][0]
        for j in range(D // 16):
            sl = pl.ds(j * 16, 16)
            r0_v[i, sl] = r0_v[i, sl] * w0 + r1_v[i, sl] * w1
        return carry

    lax.fori_loop(0, TPT, tok_loop, 0, unroll=False)
    pltpu.sync_copy(r0_v, fin_hbm.at[pl.ds(t0, TPT)])


def _combine(os, so, tw):
    mesh = plsc.VectorSubcoreMesh(**_SC_MESH)
    fn = functools.partial(
        pl.kernel,
        out_type=jax.ShapeDtypeStruct((N, D), jnp.float32),
        mesh=mesh,
        compiler_params=pltpu.CompilerParams(needs_layout_passes=False),
        scratch_types=[
            pltpu.VMEM((2 * TPT,), jnp.int32),
            pltpu.VMEM((2 * TPT + 16,), jnp.float32),
            pltpu.VMEM((TPT, D), jnp.float32),
            pltpu.VMEM((TPT, D), jnp.float32),
            pltpu.SemaphoreType.DMA,
        ],
    )(_combine_body)
    return fn(os, so, tw)


@jax.jit
def kernel(score_norm_data, W1, W3, W2, Wg, A1, B1, A3, B3, A2, B2):
    x = score_norm_data
    bf = jnp.bfloat16
    lt = _router_logits(x, Wg)
    rt, be, so, tw = _bucketize(lt)
    x_i32 = jax.lax.bitcast_convert_type(
        x.astype(bf).reshape(N, D2, 2), jnp.int32)
    xs_i = _gather_x(rt, x_i32)
    xs = jax.lax.bitcast_convert_type(xs_i, bf).reshape(S, D)
    os = _grouped_ffn(be, xs,
                      W1.astype(bf), W3.astype(bf), W2.astype(bf),
                      A1.astype(bf), B1.astype(bf), A3.astype(bf),
                      B3.astype(bf), A2.astype(bf), B2.astype(bf))
    return _combine(os, so, tw)


# speedup vs baseline: 1.3523x; 1.3523x over previous
import functools

import jax
import jax.numpy as jnp
from jax import lax
from jax.experimental import pallas as pl
from jax.experimental.pallas import tpu as pltpu
from jax.experimental.pallas import tpu_sc as plsc

N = 2048
D = 768
DFF = 2048
E = 8
R = 16

BT = 256
NB = 2 * N // BT + E
S = NB * BT

NW = 32
SPT = S // NW
PPT = 2 * N // NW
TPT = N // NW
GB = 48

_SC_MESH = dict(core_axis_name="c", subcore_axis_name="s")


def _logits_kernel(x_ref, wg_ref, lt_ref):
    lt_ref[...] = jax.lax.dot_general(
        wg_ref[...], x_ref[...], (((1,), (1,)), ((), ())),
        preferred_element_type=jnp.float32)


def _router_logits(x, Wg):
    return pl.pallas_call(
        _logits_kernel,
        out_shape=jax.ShapeDtypeStruct((E, N), jnp.float32),
    )(x, Wg)


NT16 = 16
TB = N // NT16
PB = 2 * TB


def _bucket_body(lt_hbm, rt_hbm, be_hbm, so_hbm, tw_hbm,
                 lt_v, w2_v, be_v, zero_v, tok2_v, pos2_v, cnt_sh, cnt_v,
                 cnt_all_v, sem):
    cid = lax.axis_index("c")
    sid = lax.axis_index("s")
    iota16 = lax.iota(jnp.int32, 16)

    @pl.when(cid == 0)
    def _bucket():
        t0 = sid * TB

        for j in range(S // NT16 // 16):
            zero_v[pl.ds(j * 16, 16)] = jnp.zeros((16,), jnp.int32)
        pltpu.sync_copy(zero_v, rt_hbm.at[pl.ds(sid * (S // NT16), S // NT16)])

        pltpu.sync_copy(lt_hbm.at[:, pl.ds(t0, TB)], lt_v)

        def route_chunk(c):
            base = c * 16
            ls = [lt_v[e, pl.ds(base, 16)] for e in range(E)]
            m1 = ls[0]
            i1 = jnp.zeros((16,), jnp.int32)
            for e in range(1, E):
                upd = ls[e] > m1
                m1 = jnp.where(upd, ls[e], m1)
                i1 = jnp.where(upd, e, i1)
            m2 = jnp.full((16,), -1e30, jnp.float32)
            i2 = jnp.zeros((16,), jnp.int32)
            for e in range(E):
                upd = (ls[e] > m2) & (i1 != e)
                m2 = jnp.where(upd, ls[e], m2)
                i2 = jnp.where(upd, e, i2)
            w1 = 1.0 / (1.0 + jnp.exp(m2 - m1))
            tok2_v[0, pl.ds(base, 16)] = i1
            tok2_v[1, pl.ds(base, 16)] = i2
            return w1

        for c in range(TB // 16):
            w1 = route_chunk(c)
            w2_v[0, pl.ds(c * 16, 16)] = w1
            w2_v[1, pl.ds(c * 16, 16)] = 1.0 - w1
        pltpu.sync_copy(w2_v.at[0], tw_hbm.at[pl.ds(t0, TB)])
        pltpu.sync_copy(w2_v.at[1], tw_hbm.at[pl.ds(N + t0, TB)])

        def count_chunk(c, cnt):
            for k in range(2):
                v = tok2_v[k, pl.ds(c * 16, 16)]
                for e in range(E):
                    pc = jnp.sum(jnp.where(v == e, 1, 0))
                    cnt = cnt + jnp.where(iota16 == e, pc, 0)
            return cnt

        cnt = lax.fori_loop(0, TB // 16, count_chunk,
                            jnp.zeros((16,), jnp.int32), unroll=False)
        cnt_v[pl.ds(0, 16)] = cnt
        pltpu.sync_copy(cnt_v, cnt_sh.at[pl.ds(sid * 16, 16)])
        plsc.subcore_barrier()

        pltpu.sync_copy(cnt_sh, cnt_all_v)
        totals = jnp.zeros((16,), jnp.int32)
        prefix = jnp.zeros((16,), jnp.int32)
        for t in range(NT16):
            row = cnt_all_v[pl.ds(t * 16, 16)]
            totals = totals + row
            prefix = prefix + jnp.where(t < sid, row, 0)
        nblk = (totals + (BT - 1)) // BT
        csum = plsc.cumsum(nblk)
        startblk = csum - nblk
        segstart = startblk * BT
        base_v = segstart + prefix

        @pl.when(sid == 0)
        def _():
            for j in range(2):
                bid = iota16 + 16 * j
                be = jnp.zeros((16,), jnp.int32)
                for e in range(E):
                    sb = jnp.sum(jnp.where(iota16 == e, startblk, 0))
                    eb = jnp.sum(jnp.where(iota16 == e, csum, 0))
                    be = jnp.where((bid >= sb) & (bid < eb), e, be)
                be_v[pl.ds(16 * j, 16)] = be
            pltpu.sync_copy(be_v, be_hbm)

        def pos_chunk(c, base_v):
            for k in range(2):
                v = tok2_v[k, pl.ds(c * 16, 16)]
                pos = jnp.zeros((16,), jnp.int32)
                for e in range(E):
                    m = v == e
                    inc = jnp.where(m, 1, 0)
                    s = plsc.cumsum(inc)
                    base_e = jnp.sum(jnp.where(iota16 == e, base_v, 0))
                    pos = jnp.where(m, base_e + s - 1, pos)
                    base_v = base_v + jnp.where(iota16 == e, jnp.sum(inc), 0)
                pos2_v[k, pl.ds(c * 16, 16)] = pos
            return base_v

        lax.fori_loop(0, TB // 16, pos_chunk, base_v, unroll=False)

        for c in range(TB // 16):
            tid = t0 + c * 16 + iota16
            tok2_v[0, pl.ds(c * 16, 16)] = tid
            tok2_v[1, pl.ds(c * 16, 16)] = tid

        pltpu.sync_copy(pos2_v.at[0], so_hbm.at[pl.ds(t0, TB)])
        pltpu.sync_copy(pos2_v.at[1], so_hbm.at[pl.ds(N + t0, TB)])

        plsc.subcore_barrier()
        pltpu.sync_copy(tok2_v.at[0], rt_hbm.at[pos2_v.at[0]])
        pltpu.sync_copy(tok2_v.at[1], rt_hbm.at[pos2_v.at[1]])


def _bucketize(lt_flat):
    mesh = plsc.VectorSubcoreMesh(**_SC_MESH)
    fn = functools.partial(
        pl.kernel,
        out_type=[
            jax.ShapeDtypeStruct((S,), jnp.int32),
            jax.ShapeDtypeStruct((NW,), jnp.int32),
            jax.ShapeDtypeStruct((2 * N,), jnp.int32),
            jax.ShapeDtypeStruct((2 * N,), jnp.float32),
        ],
        mesh=mesh,
        compiler_params=pltpu.CompilerParams(needs_layout_passes=False),
        scratch_types=[
            pltpu.VMEM((E, TB), jnp.float32),
            pltpu.VMEM((2, TB), jnp.float32),
            pltpu.VMEM((NW,), jnp.int32),
            pltpu.VMEM((S // NT16,), jnp.int32),
            pltpu.VMEM((2, TB), jnp.int32),
            pltpu.VMEM((2, TB), jnp.int32),
            pltpu.VMEM_SHARED((NT16 * 16,), jnp.int32),
            pltpu.VMEM((16,), jnp.int32),
            pltpu.VMEM((NT16 * 16,), jnp.int32),
            pltpu.SemaphoreType.DMA,
        ],
    )(_bucket_body)
    return fn(lt_flat)


def _gather_body(rt_hbm, x_hbm, xs_hbm, idx_v, buf0, buf1, buf2,
                 sem0, sem1, sem2):
    cid = lax.axis_index("c")
    sid = lax.axis_index("s")
    wid = sid * 2 + cid
    lo = wid * SPT

    pltpu.sync_copy(rt_hbm.at[pl.ds(lo, SPT)], idx_v)
    bufs = [buf0, buf1, buf2]
    sems = [sem0, sem1, sem2]
    nb = SPT // GB
    cps = [None] * nb
    for j in range(min(3, nb)):
        cps[j] = pltpu.async_copy(
            x_hbm.at[idx_v.at[pl.ds(j * GB, GB)]], bufs[j], sems[j])
    for j in range(nb):
        cps[j].wait()
        pltpu.sync_copy(bufs[j % 3], xs_hbm.at[pl.ds(lo + j * GB, GB)])
        if j + 3 < nb:
            cps[j + 3] = pltpu.async_copy(
                x_hbm.at[idx_v.at[pl.ds((j + 3) * GB, GB)]],
                bufs[j % 3], sems[j % 3])


def _gather_x(rt, x):
    mesh = plsc.VectorSubcoreMesh(**_SC_MESH)
    fn = functools.partial(
        pl.kernel,
        out_type=jax.ShapeDtypeStruct((S, D), jnp.float32),
        mesh=mesh,
        compiler_params=pltpu.CompilerParams(needs_layout_passes=False),
        scratch_types=[
            pltpu.VMEM((SPT,), jnp.int32),
            pltpu.VMEM((GB, D), jnp.float32),
            pltpu.VMEM((GB, D), jnp.float32),
            pltpu.VMEM((GB, D), jnp.float32),
            pltpu.SemaphoreType.DMA,
            pltpu.SemaphoreType.DMA,
            pltpu.SemaphoreType.DMA,
        ],
    )(_gather_body)
    return fn(rt, x)


def _bdot_t(a, b):
    return jax.lax.dot_general(a, b, (((1,), (1,)), ((), ())),
                               preferred_element_type=jnp.float32)


def _grouped_kernel(be_ref, xs_ref, w1_ref, w3_ref, w2_ref,
                    a1_ref, b1_ref, a3_ref, b3_ref, a2_ref, b2_ref,
                    a1b_ref, b1b_ref, a3b_ref, b3b_ref, a2b_ref, b2b_ref,
                    out_ref):
    las = [(a1_ref, b1_ref, a3_ref, b3_ref, a2_ref, b2_ref),
           (a1b_ref, b1b_ref, a3b_ref, b3b_ref, a2b_ref, b2b_ref)]
    for i in range(2):
        a1r, b1r, a3r, b3r, a2r, b2r = las[i]
        rows = pl.ds(i * BT, BT)
        xb = xs_ref[rows, :]
        u1 = _bdot_t(xb, a1r[0])
        w1 = _bdot_t(xb, w1_ref[...]) + _bdot_t(u1, b1r[0])
        u3 = _bdot_t(xb, a3r[0])
        w3 = _bdot_t(xb, w3_ref[...]) + _bdot_t(u3, b3r[0])
        h = w1 * jax.nn.sigmoid(w1) * w3
        u2 = _bdot_t(h, a2r[0])
        out_ref[rows, :] = _bdot_t(h, w2_ref[...]) + _bdot_t(u2, b2r[0])


def _grouped_ffn(be, xs, W1b, W3b, W2b, A1b, B1b, A3b, B3b, A2b, B2b):
    grid_spec = pltpu.PrefetchScalarGridSpec(
        num_scalar_prefetch=1,
        grid=(NB // 2,),
        in_specs=[
            pl.BlockSpec((2 * BT, D), lambda b, be: (b, 0)),
            pl.BlockSpec((DFF, D), lambda b, be: (0, 0)),
            pl.BlockSpec((DFF, D), lambda b, be: (0, 0)),
            pl.BlockSpec((D, DFF), lambda b, be: (0, 0)),
            pl.BlockSpec((1, R, D), lambda b, be: (be[2 * b], 0, 0)),
            pl.BlockSpec((1, DFF, R), lambda b, be: (be[2 * b], 0, 0)),
            pl.BlockSpec((1, R, D), lambda b, be: (be[2 * b], 0, 0)),
            pl.BlockSpec((1, DFF, R), lambda b, be: (be[2 * b], 0, 0)),
            pl.BlockSpec((1, R, DFF), lambda b, be: (be[2 * b], 0, 0)),
            pl.BlockSpec((1, D, R), lambda b, be: (be[2 * b], 0, 0)),
            pl.BlockSpec((1, R, D), lambda b, be: (be[2 * b + 1], 0, 0)),
            pl.BlockSpec((1, DFF, R), lambda b, be: (be[2 * b + 1], 0, 0)),
            pl.BlockSpec((1, R, D), lambda b, be: (be[2 * b + 1], 0, 0)),
            pl.BlockSpec((1, DFF, R), lambda b, be: (be[2 * b + 1], 0, 0)),
            pl.BlockSpec((1, R, DFF), lambda b, be: (be[2 * b + 1], 0, 0)),
            pl.BlockSpec((1, D, R), lambda b, be: (be[2 * b + 1], 0, 0)),
        ],
        out_specs=pl.BlockSpec((2 * BT, D), lambda b, be: (b, 0)),
    )
    return pl.pallas_call(
        _grouped_kernel,
        grid_spec=grid_spec,
        out_shape=jax.ShapeDtypeStruct((S, D), jnp.float32),
        compiler_params=pltpu.CompilerParams(
            dimension_semantics=("parallel",),
        ),
    )(be, xs, W1b, W3b, W2b, A1b, B1b, A3b, B3b, A2b, B2b,
      A1b, B1b, A3b, B3b, A2b, B2b)


def _combine_body(os_hbm, so_hbm, tw_hbm, fin_hbm,
                  so_v, tw_v, r0_v, r1_v, sem):
    cid = lax.axis_index("c")
    sid = lax.axis_index("s")
    wid = sid * 2 + cid
    t0 = wid * TPT

    pltpu.sync_copy(so_hbm.at[pl.ds(t0, TPT)], so_v.at[pl.ds(0, TPT)])
    pltpu.sync_copy(so_hbm.at[pl.ds(N + t0, TPT)], so_v.at[pl.ds(TPT, TPT)])
    pltpu.sync_copy(tw_hbm.at[pl.ds(t0, TPT)], tw_v.at[pl.ds(0, TPT)])
    pltpu.sync_copy(tw_hbm.at[pl.ds(N + t0, TPT)], tw_v.at[pl.ds(TPT, TPT)])

    pltpu.async_copy(os_hbm.at[so_v.at[pl.ds(0, TPT)]], r0_v, sem).wait()
    pltpu.async_copy(os_hbm.at[so_v.at[pl.ds(TPT, TPT)]], r1_v, sem).wait()

    def tok_loop(i, carry):
        w0 = tw_v[pl.ds(i, 16)][0]
        w1 = tw_v[pl.ds(TPT + i, 16)][0]
        for j in range(D // 16):
            sl = pl.ds(j * 16, 16)
            r0_v[i, sl] = r0_v[i, sl] * w0 + r1_v[i, sl] * w1
        return carry

    lax.fori_loop(0, TPT, tok_loop, 0, unroll=False)
    pltpu.sync_copy(r0_v, fin_hbm.at[pl.ds(t0, TPT)])


def _combine(os, so, tw):
    mesh = plsc.VectorSubcoreMesh(**_SC_MESH)
    fn = functools.partial(
        pl.kernel,
        out_type=jax.ShapeDtypeStruct((N, D), jnp.float32),
        mesh=mesh,
        compiler_params=pltpu.CompilerParams(needs_layout_passes=False),
        scratch_types=[
            pltpu.VMEM((2 * TPT,), jnp.int32),
            pltpu.VMEM((2 * TPT + 16,), jnp.float32),
            pltpu.VMEM((TPT, D), jnp.float32),
            pltpu.VMEM((TPT, D), jnp.float32),
            pltpu.SemaphoreType.DMA,
        ],
    )(_combine_body)
    return fn(os, so, tw)


@jax.jit
def kernel(score_norm_data, W1, W3, W2, Wg, A1, B1, A3, B3, A2, B2):
    x = score_norm_data
    bf = jnp.bfloat16
    lt = _router_logits(x, Wg)
    rt, be, so, tw = _bucketize(lt)
    xs = _gather_x(rt, x)
    os = _grouped_ffn(be, xs, W1, W3, W2, A1, B1, A3, B3, A2, B2)
    return _combine(os, so, tw)


# speedup vs baseline: 1.4413x; 1.0658x over previous
import functools

import jax
import jax.numpy as jnp
from jax import lax
from jax.experimental import pallas as pl
from jax.experimental.pallas import tpu as pltpu
from jax.experimental.pallas import tpu_sc as plsc

N = 2048
D = 768
DFF = 2048
E = 8
R = 16

BT = 256
NB = 2 * N // BT + E
S = NB * BT

NW = 32
SPT = S // NW
PPT = 2 * N // NW
TPT = N // NW
GB = 48

_SC_MESH = dict(core_axis_name="c", subcore_axis_name="s")


def _rne16(v):
    u = jax.lax.bitcast_convert_type(v, jnp.int32)
    return u + jnp.int32(0x7FFF) + ((u >> 16) & 1)


def _logits_kernel(x_ref, wg_ref, lt_ref, xi_ref):
    lt_ref[...] = jax.lax.dot_general(
        wg_ref[...], x_ref[...], (((1,), (1,)), ((), ())),
        preferred_element_type=jnp.float32)
    lo = jax.lax.shift_right_logical(_rne16(x_ref[:, :D // 2]), 16)
    hi = _rne16(x_ref[:, D // 2:]) & jnp.int32(-65536)
    xi_ref[...] = lo | hi


def _router_logits(x, Wg):
    return pl.pallas_call(
        _logits_kernel,
        out_shape=[
            jax.ShapeDtypeStruct((E, N), jnp.float32),
            jax.ShapeDtypeStruct((N, D // 2), jnp.int32),
        ],
    )(x, Wg)


NT16 = 16
TB = N // NT16
PB = 2 * TB


def _bucket_body(lt_hbm, rt_hbm, be_hbm, so_hbm, tw_hbm,
                 lt_v, w2_v, be_v, zero_v, tok2_v, pos2_v, cnt_sh, cnt_v,
                 cnt_all_v, sem):
    cid = lax.axis_index("c")
    sid = lax.axis_index("s")
    iota16 = lax.iota(jnp.int32, 16)

    @pl.when(cid == 0)
    def _bucket():
        t0 = sid * TB

        for j in range(S // NT16 // 16):
            zero_v[pl.ds(j * 16, 16)] = jnp.zeros((16,), jnp.int32)
        pltpu.sync_copy(zero_v, rt_hbm.at[pl.ds(sid * (S // NT16), S // NT16)])

        pltpu.sync_copy(lt_hbm.at[:, pl.ds(t0, TB)], lt_v)

        def route_chunk(c):
            base = c * 16
            ls = [lt_v[e, pl.ds(base, 16)] for e in range(E)]
            m1 = ls[0]
            i1 = jnp.zeros((16,), jnp.int32)
            for e in range(1, E):
                upd = ls[e] > m1
                m1 = jnp.where(upd, ls[e], m1)
                i1 = jnp.where(upd, e, i1)
            m2 = jnp.full((16,), -1e30, jnp.float32)
            i2 = jnp.zeros((16,), jnp.int32)
            for e in range(E):
                upd = (ls[e] > m2) & (i1 != e)
                m2 = jnp.where(upd, ls[e], m2)
                i2 = jnp.where(upd, e, i2)
            w1 = 1.0 / (1.0 + jnp.exp(m2 - m1))
            tok2_v[0, pl.ds(base, 16)] = i1
            tok2_v[1, pl.ds(base, 16)] = i2
            return w1

        for c in range(TB // 16):
            w1 = route_chunk(c)
            w2_v[0, pl.ds(c * 16, 16)] = w1
            w2_v[1, pl.ds(c * 16, 16)] = 1.0 - w1
        pltpu.sync_copy(w2_v.at[0], tw_hbm.at[pl.ds(t0, TB)])
        pltpu.sync_copy(w2_v.at[1], tw_hbm.at[pl.ds(N + t0, TB)])

        def count_chunk(c, cnt):
            for k in range(2):
                v = tok2_v[k, pl.ds(c * 16, 16)]
                for e in range(E):
                    pc = jnp.sum(jnp.where(v == e, 1, 0))
                    cnt = cnt + jnp.where(iota16 == e, pc, 0)
            return cnt

        cnt = lax.fori_loop(0, TB // 16, count_chunk,
                            jnp.zeros((16,), jnp.int32), unroll=False)
        cnt_v[pl.ds(0, 16)] = cnt
        pltpu.sync_copy(cnt_v, cnt_sh.at[pl.ds(sid * 16, 16)])
        plsc.subcore_barrier()

        pltpu.sync_copy(cnt_sh, cnt_all_v)
        totals = jnp.zeros((16,), jnp.int32)
        prefix = jnp.zeros((16,), jnp.int32)
        for t in range(NT16):
            row = cnt_all_v[pl.ds(t * 16, 16)]
            totals = totals + row
            prefix = prefix + jnp.where(t < sid, row, 0)
        nblk = (totals + (BT - 1)) // BT
        csum = plsc.cumsum(nblk)
        startblk = csum - nblk
        segstart = startblk * BT
        base_v = segstart + prefix

        @pl.when(sid == 0)
        def _():
            for j in range(2):
                bid = iota16 + 16 * j
                be = jnp.zeros((16,), jnp.int32)
                for e in range(E):
                    sb = jnp.sum(jnp.where(iota16 == e, startblk, 0))
                    eb = jnp.sum(jnp.where(iota16 == e, csum, 0))
                    be = jnp.where((bid >= sb) & (bid < eb), e, be)
                be_v[pl.ds(16 * j, 16)] = be
            pltpu.sync_copy(be_v, be_hbm)

        def pos_chunk(c, base_v):
            for k in range(2):
                v = tok2_v[k, pl.ds(c * 16, 16)]
                pos = jnp.zeros((16,), jnp.int32)
                for e in range(E):
                    m = v == e
                    inc = jnp.where(m, 1, 0)
                    s = plsc.cumsum(inc)
                    base_e = jnp.sum(jnp.where(iota16 == e, base_v, 0))
                    pos = jnp.where(m, base_e + s - 1, pos)
                    base_v = base_v + jnp.where(iota16 == e, jnp.sum(inc), 0)
                pos2_v[k, pl.ds(c * 16, 16)] = pos
            return base_v

        lax.fori_loop(0, TB // 16, pos_chunk, base_v, unroll=False)

        for c in range(TB // 16):
            tid = t0 + c * 16 + iota16
            tok2_v[0, pl.ds(c * 16, 16)] = tid
            tok2_v[1, pl.ds(c * 16, 16)] = tid

        pltpu.sync_copy(pos2_v.at[0], so_hbm.at[pl.ds(t0, TB)])
        pltpu.sync_copy(pos2_v.at[1], so_hbm.at[pl.ds(N + t0, TB)])

        plsc.subcore_barrier()
        pltpu.sync_copy(tok2_v.at[0], rt_hbm.at[pos2_v.at[0]])
        pltpu.sync_copy(tok2_v.at[1], rt_hbm.at[pos2_v.at[1]])


def _bucketize(lt_flat):
    mesh = plsc.VectorSubcoreMesh(**_SC_MESH)
    fn = functools.partial(
        pl.kernel,
        out_type=[
            jax.ShapeDtypeStruct((S,), jnp.int32),
            jax.ShapeDtypeStruct((NW,), jnp.int32),
            jax.ShapeDtypeStruct((2 * N,), jnp.int32),
            jax.ShapeDtypeStruct((2 * N,), jnp.float32),
        ],
        mesh=mesh,
        compiler_params=pltpu.CompilerParams(needs_layout_passes=False),
        scratch_types=[
            pltpu.VMEM((E, TB), jnp.float32),
            pltpu.VMEM((2, TB), jnp.float32),
            pltpu.VMEM((NW,), jnp.int32),
            pltpu.VMEM((S // NT16,), jnp.int32),
            pltpu.VMEM((2, TB), jnp.int32),
            pltpu.VMEM((2, TB), jnp.int32),
            pltpu.VMEM_SHARED((NT16 * 16,), jnp.int32),
            pltpu.VMEM((16,), jnp.int32),
            pltpu.VMEM((NT16 * 16,), jnp.int32),
            pltpu.SemaphoreType.DMA,
        ],
    )(_bucket_body)
    return fn(lt_flat)


def _gather_body(rt_hbm, x_hbm, xs_hbm, idx_v, buf0, buf1, buf2,
                 sem0, sem1, sem2):
    cid = lax.axis_index("c")
    sid = lax.axis_index("s")
    wid = sid * 2 + cid
    lo = wid * SPT

    pltpu.sync_copy(rt_hbm.at[pl.ds(lo, SPT)], idx_v)
    bufs = [buf0, buf1, buf2]
    sems = [sem0, sem1, sem2]
    nb = SPT // GB
    cps = [None] * nb
    for j in range(min(3, nb)):
        cps[j] = pltpu.async_copy(
            x_hbm.at[idx_v.at[pl.ds(j * GB, GB)]], bufs[j], sems[j])
    for j in range(nb):
        cps[j].wait()
        pltpu.sync_copy(bufs[j % 3], xs_hbm.at[pl.ds(lo + j * GB, GB)])
        if j + 3 < nb:
            cps[j + 3] = pltpu.async_copy(
                x_hbm.at[idx_v.at[pl.ds((j + 3) * GB, GB)]],
                bufs[j % 3], sems[j % 3])


def _gather_x(rt, xi):
    mesh = plsc.VectorSubcoreMesh(**_SC_MESH)
    fn = functools.partial(
        pl.kernel,
        out_type=jax.ShapeDtypeStruct((S, D // 2), jnp.int32),
        mesh=mesh,
        compiler_params=pltpu.CompilerParams(needs_layout_passes=False),
        scratch_types=[
            pltpu.VMEM((SPT,), jnp.int32),
            pltpu.VMEM((GB, D // 2), jnp.int32),
            pltpu.VMEM((GB, D // 2), jnp.int32),
            pltpu.VMEM((GB, D // 2), jnp.int32),
            pltpu.SemaphoreType.DMA,
            pltpu.SemaphoreType.DMA,
            pltpu.SemaphoreType.DMA,
        ],
    )(_gather_body)
    return fn(rt, xi)


def _bdot_t(a, b):
    return jax.lax.dot_general(a, b, (((1,), (1,)), ((), ())),
                               preferred_element_type=jnp.float32)


def _grouped_kernel(be_ref, xs_ref, w1_ref, w3_ref, w2_ref,
                    a1_ref, b1_ref, a3_ref, b3_ref, a2_ref, b2_ref,
                    a1b_ref, b1b_ref, a3b_ref, b3b_ref, a2b_ref, b2b_ref,
                    out_ref):
    las = [(a1_ref, b1_ref, a3_ref, b3_ref, a2_ref, b2_ref),
           (a1b_ref, b1b_ref, a3b_ref, b3b_ref, a2b_ref, b2b_ref)]
    for i in range(2):
        a1r, b1r, a3r, b3r, a2r, b2r = las[i]
        rows = pl.ds(i * BT, BT)
        xi = xs_ref[rows, :]
        xlo = jax.lax.bitcast_convert_type(xi << 16, jnp.float32)
        xhi = jax.lax.bitcast_convert_type(xi & jnp.int32(-65536), jnp.float32)
        xb = jnp.concatenate([xlo, xhi], axis=1)
        u1 = _bdot_t(xb, a1r[0])
        w1 = _bdot_t(xb, w1_ref[...]) + _bdot_t(u1, b1r[0])
        u3 = _bdot_t(xb, a3r[0])
        w3 = _bdot_t(xb, w3_ref[...]) + _bdot_t(u3, b3r[0])
        h = w1 * jax.nn.sigmoid(w1) * w3
        u2 = _bdot_t(h, a2r[0])
        out_ref[rows, :] = _bdot_t(h, w2_ref[...]) + _bdot_t(u2, b2r[0])


def _grouped_ffn(be, xs, W1b, W3b, W2b, A1b, B1b, A3b, B3b, A2b, B2b):
    grid_spec = pltpu.PrefetchScalarGridSpec(
        num_scalar_prefetch=1,
        grid=(NB // 2,),
        in_specs=[
            pl.BlockSpec((2 * BT, D // 2), lambda b, be: (b, 0)),
            pl.BlockSpec((DFF, D), lambda b, be: (0, 0)),
            pl.BlockSpec((DFF, D), lambda b, be: (0, 0)),
            pl.BlockSpec((D, DFF), lambda b, be: (0, 0)),
            pl.BlockSpec((1, R, D), lambda b, be: (be[2 * b], 0, 0)),
            pl.BlockSpec((1, DFF, R), lambda b, be: (be[2 * b], 0, 0)),
            pl.BlockSpec((1, R, D), lambda b, be: (be[2 * b], 0, 0)),
            pl.BlockSpec((1, DFF, R), lambda b, be: (be[2 * b], 0, 0)),
            pl.BlockSpec((1, R, DFF), lambda b, be: (be[2 * b], 0, 0)),
            pl.BlockSpec((1, D, R), lambda b, be: (be[2 * b], 0, 0)),
            pl.BlockSpec((1, R, D), lambda b, be: (be[2 * b + 1], 0, 0)),
            pl.BlockSpec((1, DFF, R), lambda b, be: (be[2 * b + 1], 0, 0)),
            pl.BlockSpec((1, R, D), lambda b, be: (be[2 * b + 1], 0, 0)),
            pl.BlockSpec((1, DFF, R), lambda b, be: (be[2 * b + 1], 0, 0)),
            pl.BlockSpec((1, R, DFF), lambda b, be: (be[2 * b + 1], 0, 0)),
            pl.BlockSpec((1, D, R), lambda b, be: (be[2 * b + 1], 0, 0)),
        ],
        out_specs=pl.BlockSpec((2 * BT, D), lambda b, be: (b, 0)),
    )
    return pl.pallas_call(
        _grouped_kernel,
        grid_spec=grid_spec,
        out_shape=jax.ShapeDtypeStruct((S, D), jnp.float32),
        compiler_params=pltpu.CompilerParams(
            dimension_semantics=("parallel",),
        ),
    )(be, xs, W1b, W3b, W2b, A1b, B1b, A3b, B3b, A2b, B2b,
      A1b, B1b, A3b, B3b, A2b, B2b)


def _combine_body(os_hbm, so_hbm, tw_hbm, fin_hbm,
                  so_v, tw_v, r0_v, r1_v, sem):
    cid = lax.axis_index("c")
    sid = lax.axis_index("s")
    wid = sid * 2 + cid
    t0 = wid * TPT

    pltpu.sync_copy(so_hbm.at[pl.ds(t0, TPT)], so_v.at[pl.ds(0, TPT)])
    pltpu.sync_copy(so_hbm.at[pl.ds(N + t0, TPT)], so_v.at[pl.ds(TPT, TPT)])
    pltpu.sync_copy(tw_hbm.at[pl.ds(t0, TPT)], tw_v.at[pl.ds(0, TPT)])
    pltpu.sync_copy(tw_hbm.at[pl.ds(N + t0, TPT)], tw_v.at[pl.ds(TPT, TPT)])

    pltpu.async_copy(os_hbm.at[so_v.at[pl.ds(0, TPT)]], r0_v, sem).wait()
    pltpu.async_copy(os_hbm.at[so_v.at[pl.ds(TPT, TPT)]], r1_v, sem).wait()

    def tok_loop(i, carry):
        w0 = tw_v[pl.ds(i, 16)][0]
        w1 = tw_v[pl.ds(TPT + i, 16)][0]
        for j in range(D // 16):
            sl = pl.ds(j * 16, 16)
            r0_v[i, sl] = r0_v[i, sl] * w0 + r1_v[i, sl] * w1
        return carry

    lax.fori_loop(0, TPT, tok_loop, 0, unroll=False)
    pltpu.sync_copy(r0_v, fin_hbm.at[pl.ds(t0, TPT)])


def _combine(os, so, tw):
    mesh = plsc.VectorSubcoreMesh(**_SC_MESH)
    fn = functools.partial(
        pl.kernel,
        out_type=jax.ShapeDtypeStruct((N, D), jnp.float32),
        mesh=mesh,
        compiler_params=pltpu.CompilerParams(needs_layout_passes=False),
        scratch_types=[
            pltpu.VMEM((2 * TPT,), jnp.int32),
            pltpu.VMEM((2 * TPT + 16,), jnp.float32),
            pltpu.VMEM((TPT, D), jnp.float32),
            pltpu.VMEM((TPT, D), jnp.float32),
            pltpu.SemaphoreType.DMA,
        ],
    )(_combine_body)
    return fn(os, so, tw)


@jax.jit
def kernel(score_norm_data, W1, W3, W2, Wg, A1, B1, A3, B3, A2, B2):
    x = score_norm_data
    bf = jnp.bfloat16
    lt, xi = _router_logits(x, Wg)
    rt, be, so, tw = _bucketize(lt)
    xs = _gather_x(rt, xi)
    os = _grouped_ffn(be, xs, W1, W3, W2, A1, B1, A3, B3, A2, B2)
    return _combine(os, so, tw)
